# Initial kernel scaffold; baseline (speedup 1.0000x reference)
import jax
import jax.numpy as jnp
from jax.experimental import pallas as pl


def kernel(smpl_vertices, occ_volume, smpl_vertex_code, smpl_face_indices):
    raise NotImplementedError("write your pallas kernel here")


# speedup vs baseline: 298.2244x; 298.2244x over previous
import jax
import jax.numpy as jnp
from jax.experimental import pallas as pl
from jax.experimental.pallas import tpu as pltpu

XR, YR, ZR = 128, 192, 128
VOX = 2.0 / 192.0
SIG = 2.0 / 192.0
INV2S2 = 1.0 / (2.0 * SIG * SIG)
NV = 6890
VPAD = 6912
XBLK = 8


def _axis_weights(viota_mask, coord_vec, n, axis_of_v):
    grid_f = coord_vec / VOX + (0.5 * n - 0.5)
    base = jnp.floor(grid_f)
    if axis_of_v == 1:
        idx = jax.lax.broadcasted_iota(jnp.int32, (n, 1), 0).astype(jnp.float32)
    else:
        idx = jax.lax.broadcasted_iota(jnp.int32, (1, n), 1).astype(jnp.float32)
    center = (idx + (0.5 - 0.5 * n)) * VOX
    d = center - coord_vec
    w = jnp.exp(-(d * d) * INV2S2)
    mask = (idx >= base - 3.0) & (idx <= base + 3.0) & viota_mask
    return w * mask.astype(jnp.float32)


def _tables_kernel(vx_ref, vy_ref, vz_ref, code_ref, wxt_ref, wyt_ref, b_ref):
    vmask_l = jax.lax.broadcasted_iota(jnp.int32, (1, VPAD), 1) < NV
    wxt_ref[...] = _axis_weights(vmask_l, vx_ref[...], XR, axis_of_v=1)
    wyt_ref[...] = _axis_weights(vmask_l, vy_ref[...], YR, axis_of_v=1)
    vmask_s = jax.lax.broadcasted_iota(jnp.int32, (VPAD, 1), 0) < NV
    wz = _axis_weights(vmask_s, vz_ref[...], ZR, axis_of_v=0)
    for c in range(4):
        b_ref[:, c * ZR:(c + 1) * ZR] = wz * code_ref[:, c:c + 1]


def _accum_kernel(wxt_ref, wyt_ref, b_ref, occ_ref, out_ref):
    wyt = wyt_ref[...]
    bmat = b_ref[...]
    for x in range(XBLK):
        row = wxt_ref[x:x + 1, :]
        mt = wyt * row
        acc = jax.lax.dot_general(
            mt, bmat, (((1,), (0,)), ((), ())),
            preferred_element_type=jnp.float32)
        gate = (occ_ref[x] > 1e-3).astype(jnp.float32)
        for c in range(4):
            sl = acc[:, c * ZR:(c + 1) * ZR] * gate
            if c == 3:
                sl = sl + 1e-3
            out_ref[x, :, c * ZR:(c + 1) * ZR] = sl


def kernel(smpl_vertices, occ_volume, smpl_vertex_code, smpl_face_indices):
    del smpl_face_indices
    pad = VPAD - NV
    verts = jnp.pad(smpl_vertices, ((0, pad), (0, 0)))
    code4 = jnp.pad(smpl_vertex_code, ((0, pad), (0, 1)),
                    constant_values=1.0)
    vx = verts[:, 0].reshape(1, VPAD)
    vy = verts[:, 1].reshape(1, VPAD)
    vz = verts[:, 2].reshape(VPAD, 1)

    wxt, wyt, bmat = pl.pallas_call(
        _tables_kernel,
        out_shape=[
            jax.ShapeDtypeStruct((XR, VPAD), jnp.float32),
            jax.ShapeDtypeStruct((YR, VPAD), jnp.float32),
            jax.ShapeDtypeStruct((VPAD, 4 * ZR), jnp.float32),
        ],
    )(vx, vy, vz, code4)

    out = pl.pallas_call(
        _accum_kernel,
        grid=(XR // XBLK,),
        in_specs=[
            pl.BlockSpec((XBLK, VPAD), lambda i: (i, 0)),
            pl.BlockSpec((YR, VPAD), lambda i: (0, 0)),
            pl.BlockSpec((VPAD, 4 * ZR), lambda i: (0, 0)),
            pl.BlockSpec((XBLK, YR, ZR), lambda i: (i, 0, 0)),
        ],
        out_specs=pl.BlockSpec((XBLK, YR, 4 * ZR), lambda i: (i, 0, 0)),
        out_shape=jax.ShapeDtypeStruct((XR, YR, 4 * ZR), jnp.float32),
    )(wxt, wyt, bmat, occ_volume)

    out4 = out.reshape(XR, YR, 4, ZR)
    semantic_volume = jnp.transpose(out4[:, :, :3, :], (0, 1, 3, 2))
    weight_sum_volume = out4[:, :, 3, :]
    return semantic_volume, weight_sum_volume
